# baseline (speedup 1.0000x reference)
import functools

import jax
import jax.numpy as jnp
from jax.experimental import pallas as pl
from jax.experimental.pallas import tpu as pltpu


def _contract_last(lhs, rhs):
    return jax.lax.dot_general(lhs, rhs, (((1,), (1,)), ((), ())),
                               precision=jax.lax.Precision.HIGHEST,
                               preferred_element_type=jnp.float32)


def _fused_kernel(x_ref, w1_ref, g_ref, c_ref, w2_ref, b2_ref, e_ref, o_ref,
                  *, hw):
    x = x_ref[...]
    s = (jnp.sum(x, axis=0) * (1.0 / hw)
         + jnp.max(x, axis=0))
    h = _contract_last(s, w1_ref[...]) * g_ref[...] + c_ref[...]
    h = jnp.maximum(h, 0.0)
    logits = _contract_last(h, w2_ref[...]) + b2_ref[...]
    m = jnp.max(logits, axis=1, keepdims=True)
    p = jnp.exp(logits - m)
    a = p / jnp.sum(p, axis=1, keepdims=True)
    scale = jnp.dot(a, e_ref[...], precision=jax.lax.Precision.HIGHEST,
                    preferred_element_type=jnp.float32)
    o_ref[...] = (x * scale[None, :, :]).astype(o_ref.dtype)


def kernel(x, w1, b1, gamma, beta, run_mean, run_var, w2, b2):
    eps = 1e-5
    B, C, H, W = x.shape
    inter = w1.shape[0]
    groups = w2.shape[0]
    cpg = C // groups
    hw = H * W

    g = (gamma / jnp.sqrt(run_var + eps)).reshape(1, inter).astype(jnp.float32)
    c = (g * (b1 - run_mean).reshape(1, inter)
         + beta.reshape(1, inter)).astype(jnp.float32)
    b2r = b2.reshape(1, groups).astype(jnp.float32)
    Et = (jnp.arange(groups)[:, None] == jnp.arange(C)[None, :] // cpg
          ).astype(jnp.float32)

    xt = jnp.transpose(x, (2, 3, 0, 1)).reshape(hw, B, C)
    bblk = next(d for d in (16, 8, 4, 2, 1) if B % d == 0)
    fused = functools.partial(_fused_kernel, hw=hw)
    out = pl.pallas_call(
        fused,
        out_shape=jax.ShapeDtypeStruct((hw, B, C), x.dtype),
        grid=(B // bblk,),
        in_specs=[
            pl.BlockSpec((hw, bblk, C), lambda b: (0, b, 0)),
            pl.BlockSpec((inter, C), lambda b: (0, 0)),
            pl.BlockSpec((1, inter), lambda b: (0, 0)),
            pl.BlockSpec((1, inter), lambda b: (0, 0)),
            pl.BlockSpec((groups, inter), lambda b: (0, 0)),
            pl.BlockSpec((1, groups), lambda b: (0, 0)),
            pl.BlockSpec((groups, C), lambda b: (0, 0)),
        ],
        out_specs=pl.BlockSpec((hw, bblk, C), lambda b: (0, b, 0)),
        compiler_params=pltpu.CompilerParams(
            dimension_semantics=("parallel",),
            vmem_limit_bytes=64 * 1024 * 1024),
    )(xt, w1.astype(jnp.float32), g, c, w2.astype(jnp.float32), b2r, Et)
    return jnp.transpose(out.reshape(H, W, B, C), (2, 3, 0, 1))


# speedup vs baseline: 1.0059x; 1.0059x over previous
import functools

import jax
import jax.numpy as jnp
from jax.experimental import pallas as pl
from jax.experimental.pallas import tpu as pltpu


def _contract_last(lhs, rhs):
    return jax.lax.dot_general(lhs, rhs, (((1,), (1,)), ((), ())),
                               preferred_element_type=jnp.float32)


def _fused_kernel(x_ref, w1_ref, c_ref, w2_ref, b2_ref, e_ref, o_ref, *, hw):
    x = x_ref[...]
    s = (jnp.sum(x, axis=0) * (1.0 / hw)
         + jnp.max(x, axis=0))
    h = _contract_last(s, w1_ref[...]) + c_ref[...]
    h = jnp.maximum(h, 0.0)
    logits = _contract_last(h, w2_ref[...]) + b2_ref[...]
    m = jnp.max(logits, axis=1, keepdims=True)
    p = jnp.exp(logits - m)
    a = p / jnp.sum(p, axis=1, keepdims=True)
    scale = jnp.dot(a, e_ref[...], precision=jax.lax.Precision.HIGHEST,
                    preferred_element_type=jnp.float32)
    o_ref[...] = (x * scale[None, :, :]).astype(o_ref.dtype)


def kernel(x, w1, b1, gamma, beta, run_mean, run_var, w2, b2):
    eps = 1e-5
    B, C, H, W = x.shape
    inter = w1.shape[0]
    groups = w2.shape[0]
    cpg = C // groups
    hw = H * W

    g = gamma / jnp.sqrt(run_var + eps)
    w1f = (w1 * g[:, None]).astype(jnp.float32)
    c = (g * (b1 - run_mean) + beta).reshape(1, inter).astype(jnp.float32)
    b2r = b2.reshape(1, groups).astype(jnp.float32)
    Et = (jnp.arange(groups)[:, None] == jnp.arange(C)[None, :] // cpg
          ).astype(jnp.float32)

    xt = jnp.transpose(x, (2, 3, 0, 1)).reshape(hw, B, C)
    bblk = next(d for d in (16, 8, 4, 2, 1) if B % d == 0)
    fused = functools.partial(_fused_kernel, hw=hw)
    out = pl.pallas_call(
        fused,
        out_shape=jax.ShapeDtypeStruct((hw, B, C), x.dtype),
        grid=(B // bblk,),
        in_specs=[
            pl.BlockSpec((hw, bblk, C), lambda b: (0, b, 0)),
            pl.BlockSpec((inter, C), lambda b: (0, 0)),
            pl.BlockSpec((1, inter), lambda b: (0, 0)),
            pl.BlockSpec((groups, inter), lambda b: (0, 0)),
            pl.BlockSpec((1, groups), lambda b: (0, 0)),
            pl.BlockSpec((groups, C), lambda b: (0, 0)),
        ],
        out_specs=pl.BlockSpec((hw, bblk, C), lambda b: (0, b, 0)),
        compiler_params=pltpu.CompilerParams(
            dimension_semantics=("parallel",),
            vmem_limit_bytes=64 * 1024 * 1024),
    )(xt, w1f, c, w2.astype(jnp.float32), b2r, Et)
    return jnp.transpose(out.reshape(H, W, B, C), (2, 3, 0, 1))


# speedup vs baseline: 1.0088x; 1.0029x over previous
import functools

import jax
import jax.numpy as jnp
import numpy as np
from jax.experimental import pallas as pl
from jax.experimental.pallas import tpu as pltpu


def _contract_last(lhs, rhs):
    return jax.lax.dot_general(lhs, rhs, (((1,), (1,)), ((), ())),
                               preferred_element_type=jnp.float32)


def _fused_kernel(x_ref, w1_ref, g_ref, c_ref, w2_ref, b2_ref, e_ref, o_ref,
                  *, hw):
    x = x_ref[...]
    s = (jnp.sum(x, axis=0) * (1.0 / hw)
         + jnp.max(x, axis=0))
    h = _contract_last(s, w1_ref[...] * g_ref[...]) + c_ref[...]
    h = jnp.maximum(h, 0.0)
    logits = _contract_last(h, w2_ref[...]) + b2_ref[...]
    m = jnp.max(logits, axis=1, keepdims=True)
    p = jnp.exp(logits - m)
    a = p / jnp.sum(p, axis=1, keepdims=True)
    scale = jnp.dot(a, e_ref[...], precision=jax.lax.Precision.HIGHEST,
                    preferred_element_type=jnp.float32)
    o_ref[...] = (x * scale[None, :, :]).astype(o_ref.dtype)


def kernel(x, w1, b1, gamma, beta, run_mean, run_var, w2, b2):
    eps = 1e-5
    B, C, H, W = x.shape
    inter = w1.shape[0]
    groups = w2.shape[0]
    cpg = C // groups
    hw = H * W

    g = gamma / jnp.sqrt(run_var + eps)
    gcol = g.reshape(inter, 1).astype(jnp.float32)
    c = (g * (b1 - run_mean) + beta).reshape(1, inter).astype(jnp.float32)
    b2r = b2.reshape(1, groups).astype(jnp.float32)
    Et = jnp.asarray(np.arange(groups)[:, None] == np.arange(C)[None, :] // cpg,
                     dtype=np.float32)

    xt = jnp.transpose(x, (2, 3, 0, 1)).reshape(hw, B, C)
    bblk = next(d for d in (16, 8, 4, 2, 1) if B % d == 0)
    fused = functools.partial(_fused_kernel, hw=hw)
    out = pl.pallas_call(
        fused,
        out_shape=jax.ShapeDtypeStruct((hw, B, C), x.dtype),
        grid=(B // bblk,),
        in_specs=[
            pl.BlockSpec((hw, bblk, C), lambda b: (0, b, 0)),
            pl.BlockSpec((inter, C), lambda b: (0, 0)),
            pl.BlockSpec((inter, 1), lambda b: (0, 0)),
            pl.BlockSpec((1, inter), lambda b: (0, 0)),
            pl.BlockSpec((groups, inter), lambda b: (0, 0)),
            pl.BlockSpec((1, groups), lambda b: (0, 0)),
            pl.BlockSpec((groups, C), lambda b: (0, 0)),
        ],
        out_specs=pl.BlockSpec((hw, bblk, C), lambda b: (0, b, 0)),
        compiler_params=pltpu.CompilerParams(
            dimension_semantics=("parallel",),
            vmem_limit_bytes=64 * 1024 * 1024),
    )(xt, w1.astype(jnp.float32), gcol, c, w2.astype(jnp.float32), b2r, Et)
    return jnp.transpose(out.reshape(H, W, B, C), (2, 3, 0, 1))
